# baseline (speedup 1.0000x reference)
import functools

import jax
import jax.numpy as jnp
from jax import lax
from jax.experimental import pallas as pl
from jax.experimental.pallas import tpu as pltpu
from jax.experimental.pallas import tpu_sc as plsc

N_NODES = 10000
N_PAD = 10240
CH = 128
E = 320000
NC = 2
NS = 16
NW = NC * NS
CHUNK = 128
NCH = 80
EPW = NCH * CHUNK
E_PAD = NW * EPW
NBUF = 4
RPT = N_PAD // NS

_mesh = plsc.VectorSubcoreMesh(core_axis_name="c", subcore_axis_name="s")


@functools.partial(
    pl.kernel,
    out_type=jax.ShapeDtypeStruct((NC, N_PAD), jnp.float32),
    mesh=_mesh,
    scratch_types=[
        pltpu.VMEM((NCH, CHUNK), jnp.int32),
        pltpu.VMEM((CHUNK,), jnp.float32),
        pltpu.VMEM_SHARED((N_PAD,), jnp.float32),
        pltpu.SemaphoreType.DMA,
    ],
)
def _deg_call(dst2_hbm, out_hbm, didx, buf, acc, sem):
    cid = lax.axis_index("c")
    sid = lax.axis_index("s")
    w = cid * NS + sid

    pltpu.sync_copy(dst2_hbm.at[pl.ds(w * NCH, NCH)], didx)

    @pl.loop(0, CHUNK // 16)
    def _(i):
        buf[pl.ds(i * 16, 16)] = jnp.zeros((16,), jnp.float32)

    base = sid * RPT
    for j in range(RPT // CHUNK):
        pltpu.sync_copy(buf, acc.at[pl.ds(base + j * CHUNK, CHUNK)])

    @pl.loop(0, CHUNK // 16)
    def _(i):
        buf[pl.ds(i * 16, 16)] = jnp.ones((16,), jnp.float32)

    plsc.subcore_barrier()

    @pl.loop(0, NCH)
    def _(i):
        pltpu.async_copy(buf, acc.at[didx.at[i]], sem, add=True)

    @pl.loop(0, NCH)
    def _(i):
        pltpu.make_async_copy(buf, acc.at[didx.at[0]], sem).wait()

    plsc.subcore_barrier()
    pltpu.sync_copy(acc.at[pl.ds(base, RPT)], out_hbm.at[cid].at[pl.ds(base, RPT)])


@functools.partial(
    pl.kernel,
    out_type=jax.ShapeDtypeStruct((NC, N_PAD, CH), jnp.float32),
    mesh=_mesh,
    scratch_types=[
        pltpu.VMEM((NCH, CHUNK), jnp.int32),
        pltpu.VMEM((NCH, CHUNK), jnp.int32),
        pltpu.VMEM((CHUNK,), jnp.int32),
        pltpu.VMEM((CHUNK,), jnp.int32),
        pltpu.VMEM((CHUNK, CH), jnp.float32),
        pltpu.VMEM((CHUNK, CH), jnp.float32),
        pltpu.VMEM((CHUNK, CH), jnp.float32),
        pltpu.VMEM((CHUNK, CH), jnp.float32),
        pltpu.VMEM_SHARED((N_PAD, CH), jnp.float32),
        pltpu.SemaphoreType.DMA,
        pltpu.SemaphoreType.DMA,
        pltpu.SemaphoreType.DMA,
        pltpu.SemaphoreType.DMA,
        pltpu.SemaphoreType.DMA,
        pltpu.SemaphoreType.DMA,
        pltpu.SemaphoreType.DMA,
        pltpu.SemaphoreType.DMA,
    ],
)
def _agg_call(src2_hbm, dst2_hbm, m_hbm, out_hbm, sidx, didx, sidx_e, didx_e,
              r0, r1, r2, r3, acc, g0, g1, g2, g3, s0, s1, s2, s3):
    cid = lax.axis_index("c")
    sid = lax.axis_index("s")
    w = cid * NS + sid

    pltpu.sync_copy(src2_hbm.at[pl.ds(w * NCH, NCH)], sidx)
    pltpu.sync_copy(dst2_hbm.at[pl.ds(w * NCH, NCH)], didx)

    @pl.loop(0, CHUNK)
    def _(i):
        for j in range(CH // 16):
            r0[i, pl.ds(j * 16, 16)] = jnp.zeros((16,), jnp.float32)

    base = sid * RPT
    for j in range(RPT // CHUNK):
        pltpu.sync_copy(r0, acc.at[pl.ds(base + j * CHUNK, CHUNK)])

    plsc.subcore_barrier()

    @pl.loop(0, NCH)
    def _(i):
        for j in range(CHUNK // 16):
            sidx_e[pl.ds(j * 16, 16)] = sidx[i, pl.ds(j * 16, 16)]
            didx_e[pl.ds(j * 16, 16)] = didx[i, pl.ds(j * 16, 16)]
        pltpu.async_copy(m_hbm.at[sidx_e], r0, g0).wait()
        pltpu.sync_copy(r0, acc.at[didx_e], add=True)

    plsc.subcore_barrier()
    pltpu.sync_copy(acc.at[pl.ds(base, RPT)], out_hbm.at[cid].at[pl.ds(base, RPT)])


def _k1_body(x_ref, w_ref, degs_ref, m_ref, dis_ref):
    deg = degs_ref[0] + degs_ref[1] + 1.0
    dis = lax.rsqrt(deg)
    dis_ref[...] = dis
    h = jnp.dot(x_ref[...], w_ref[...], preferred_element_type=jnp.float32)
    m_ref[...] = h * dis


def _k2_body(acc_ref, m1_ref, dis_ref, b_ref, w_ref, m2_ref):
    dis = dis_ref[...]
    a = acc_ref[0, :N_NODES, :] + acc_ref[1, :N_NODES, :] + m1_ref[...]
    h = jnp.maximum(a * dis + b_ref[...], 0.0)
    m2_ref[...] = jnp.dot(h, w_ref[...], preferred_element_type=jnp.float32) * dis


def _k3_body(acc_ref, m2_ref, dis_ref, b_ref, out_ref):
    a = acc_ref[0, :N_NODES, :] + acc_ref[1, :N_NODES, :] + m2_ref[...]
    out_ref[...] = a * dis_ref[...] + b_ref[...]


_k1 = pl.pallas_call(
    _k1_body,
    out_shape=[
        jax.ShapeDtypeStruct((N_NODES, CH), jnp.float32),
        jax.ShapeDtypeStruct((N_NODES, 1), jnp.float32),
    ],
)

_k2 = pl.pallas_call(
    _k2_body,
    out_shape=jax.ShapeDtypeStruct((N_NODES, CH), jnp.float32),
)

_k3 = pl.pallas_call(
    _k3_body,
    out_shape=jax.ShapeDtypeStruct((N_NODES, CH), jnp.float32),
)


def kernel(x, edge_index, W1, b1, W2, b2):
    ei = edge_index.astype(jnp.int32)
    npad = E_PAD - E
    src2 = jnp.concatenate([ei[0], jnp.zeros((npad,), jnp.int32)]).reshape(NW * NCH, CHUNK)
    dst_p = jnp.concatenate([ei[1], jnp.full((npad,), N_NODES, jnp.int32)])
    dst2 = dst_p.reshape(NW * NCH, CHUNK)
    b1r = b1.reshape(1, CH)
    b2r = b2.reshape(1, CH)

    degs = _deg_call(dst2)
    degs3 = degs[:, :N_NODES, None]
    m1, dis = _k1(x, W1, degs3)
    acc1 = _agg_call(src2, dst2, m1)
    m2 = _k2(acc1, m1, dis, b1r, W2)
    acc2 = _agg_call(src2, dst2, m2)
    return _k3(acc2, m2, dis, b2r)


# speedup vs baseline: 2.0989x; 2.0989x over previous
import functools

import jax
import jax.numpy as jnp
from jax import lax
from jax.experimental import pallas as pl
from jax.experimental.pallas import tpu as pltpu
from jax.experimental.pallas import tpu_sc as plsc

N_NODES = 10000
N_PAD = 10240
CH = 128
E = 320000
NC = 2
NS = 16
NW = NC * NS
EPW = E // NW
CHUNK = 128
NFULL = EPW // CHUNK
TAIL = EPW - NFULL * CHUNK
RPT = N_PAD // NS
NCH = 80
E_PAD = NW * NCH * CHUNK

_mesh = plsc.VectorSubcoreMesh(core_axis_name="c", subcore_axis_name="s")


@functools.partial(
    pl.kernel,
    out_type=jax.ShapeDtypeStruct((NC, N_PAD), jnp.float32),
    mesh=_mesh,
    scratch_types=[
        pltpu.VMEM((NCH, CHUNK), jnp.int32),
        pltpu.VMEM((CHUNK,), jnp.float32),
        pltpu.VMEM_SHARED((N_PAD,), jnp.float32),
        pltpu.SemaphoreType.DMA,
    ],
)
def _deg_call(dst2_hbm, out_hbm, didx, buf, acc, sem):
    cid = lax.axis_index("c")
    sid = lax.axis_index("s")
    w = cid * NS + sid

    pltpu.sync_copy(dst2_hbm.at[pl.ds(w * NCH, NCH)], didx)

    @pl.loop(0, CHUNK // 16)
    def _(i):
        buf[pl.ds(i * 16, 16)] = jnp.zeros((16,), jnp.float32)

    base = sid * RPT
    for j in range(RPT // CHUNK):
        pltpu.sync_copy(buf, acc.at[pl.ds(base + j * CHUNK, CHUNK)])

    @pl.loop(0, CHUNK // 16)
    def _(i):
        buf[pl.ds(i * 16, 16)] = jnp.ones((16,), jnp.float32)

    plsc.subcore_barrier()

    @pl.loop(0, NCH)
    def _(i):
        pltpu.async_copy(buf, acc.at[didx.at[i]], sem, add=True)

    @pl.loop(0, NCH)
    def _(i):
        pltpu.make_async_copy(buf, acc.at[didx.at[0]], sem).wait()

    plsc.subcore_barrier()
    pltpu.sync_copy(acc.at[pl.ds(base, RPT)], out_hbm.at[cid].at[pl.ds(base, RPT)])


@functools.partial(
    pl.kernel,
    out_type=jax.ShapeDtypeStruct((NC, N_PAD, CH), jnp.float32),
    mesh=_mesh,
    scratch_types=[
        pltpu.VMEM((CHUNK,), jnp.int32),
        pltpu.VMEM((CHUNK,), jnp.int32),
        pltpu.VMEM((TAIL,), jnp.int32),
        pltpu.VMEM((TAIL,), jnp.int32),
        pltpu.VMEM((CHUNK, CH), jnp.float32),
        pltpu.VMEM_SHARED((N_PAD, CH), jnp.float32),
        pltpu.SemaphoreType.DMA,
    ],
)
def _agg_call(src_hbm, dst_hbm, m_hbm, out_hbm, sidx, didx, sidx_t, didx_t, rows, acc, sem):
    cid = lax.axis_index("c")
    sid = lax.axis_index("s")
    ebase = (cid * NS + sid) * EPW

    @pl.loop(0, CHUNK)
    def _(i):
        for j in range(CH // 16):
            rows[i, pl.ds(j * 16, 16)] = jnp.zeros((16,), jnp.float32)

    base = sid * RPT
    for j in range(RPT // CHUNK):
        pltpu.sync_copy(rows, acc.at[pl.ds(base + j * CHUNK, CHUNK)])

    plsc.subcore_barrier()

    @pl.loop(0, NFULL)
    def _(i):
        e0 = ebase + i * CHUNK
        pltpu.sync_copy(src_hbm.at[pl.ds(e0, CHUNK)], sidx)
        pltpu.sync_copy(dst_hbm.at[pl.ds(e0, CHUNK)], didx)
        pltpu.async_copy(m_hbm.at[sidx], rows, sem).wait()
        pltpu.sync_copy(rows, acc.at[didx], add=True)

    e0 = ebase + NFULL * CHUNK
    pltpu.sync_copy(src_hbm.at[pl.ds(e0, TAIL)], sidx_t)
    pltpu.sync_copy(dst_hbm.at[pl.ds(e0, TAIL)], didx_t)
    pltpu.async_copy(m_hbm.at[sidx_t], rows.at[pl.ds(0, TAIL)], sem).wait()
    pltpu.sync_copy(rows.at[pl.ds(0, TAIL)], acc.at[didx_t], add=True)

    plsc.subcore_barrier()
    pltpu.sync_copy(acc.at[pl.ds(base, RPT)], out_hbm.at[cid].at[pl.ds(base, RPT)])


def _k1_body(x_ref, w_ref, degs_ref, m_ref, dis_ref):
    deg = degs_ref[0] + degs_ref[1] + 1.0
    dis = lax.rsqrt(deg)
    dis_ref[...] = dis
    h = jnp.dot(x_ref[...], w_ref[...], preferred_element_type=jnp.float32)
    m_ref[...] = h * dis


def _k2_body(acc_ref, m1_ref, dis_ref, b_ref, w_ref, m2_ref):
    dis = dis_ref[...]
    a = acc_ref[0, :N_NODES, :] + acc_ref[1, :N_NODES, :] + m1_ref[...]
    h = jnp.maximum(a * dis + b_ref[...], 0.0)
    m2_ref[...] = jnp.dot(h, w_ref[...], preferred_element_type=jnp.float32) * dis


def _k3_body(acc_ref, m2_ref, dis_ref, b_ref, out_ref):
    a = acc_ref[0, :N_NODES, :] + acc_ref[1, :N_NODES, :] + m2_ref[...]
    out_ref[...] = a * dis_ref[...] + b_ref[...]


_k1 = pl.pallas_call(
    _k1_body,
    out_shape=[
        jax.ShapeDtypeStruct((N_NODES, CH), jnp.float32),
        jax.ShapeDtypeStruct((N_NODES, 1), jnp.float32),
    ],
)

_k2 = pl.pallas_call(
    _k2_body,
    out_shape=jax.ShapeDtypeStruct((N_NODES, CH), jnp.float32),
)

_k3 = pl.pallas_call(
    _k3_body,
    out_shape=jax.ShapeDtypeStruct((N_NODES, CH), jnp.float32),
)


def kernel(x, edge_index, W1, b1, W2, b2):
    ei = edge_index.astype(jnp.int32)
    src = ei[0]
    dst = ei[1]
    b1r = b1.reshape(1, CH)
    b2r = b2.reshape(1, CH)

    npad = E_PAD - E
    pad_dst = N_NODES + (jnp.arange(npad, dtype=jnp.int32) % (N_PAD - N_NODES))
    dst2 = jnp.concatenate([dst, pad_dst]).reshape(NW * NCH, CHUNK)
    degs = _deg_call(dst2)
    degs3 = degs[:, :N_NODES, None]
    m1, dis = _k1(x, W1, degs3)
    acc1 = _agg_call(src, dst, m1)
    m2 = _k2(acc1, m1, dis, b1r, W2)
    acc2 = _agg_call(src, dst, m2)
    return _k3(acc2, m2, dis, b2r)
